# Initial kernel scaffold; baseline (speedup 1.0000x reference)
import jax
import jax.numpy as jnp
from jax.experimental import pallas as pl


def kernel(u_idx, v_idx, u_table, v_table):
    raise NotImplementedError("write your pallas kernel here")


# speedup vs baseline: 8.4018x; 8.4018x over previous
import jax
import jax.numpy as jnp
from jax.experimental import pallas as pl
from jax.experimental.pallas import tpu as pltpu

_TILE = 128


def _round_up(x: int, m: int) -> int:
    return (x + m - 1) // m * m


def _gmf_gather_kernel(u_ids_ref, v_ids_ref, u_tbl_ref, v_tbl_ref, out_ref):
    for mi in range(_TILE):
        u_row = u_tbl_ref[u_ids_ref[0, 0, mi], 0]
        v_row = v_tbl_ref[v_ids_ref[0, 0, mi], 0]
        out_ref[mi, 0] = u_row * v_row


@jax.jit
def kernel(u_idx, v_idx, u_table, v_table):
    batch = int(u_idx.shape[0])
    nu, emb = u_table.shape
    ni, emb_v = v_table.shape
    assert emb == emb_v, "embedding dims must match"
    out_dtype = jnp.result_type(u_table.dtype, v_table.dtype)

    u_idx = jnp.clip(u_idx.astype(jnp.int32), 0, nu - 1)
    v_idx = jnp.clip(v_idx.astype(jnp.int32), 0, ni - 1)

    batch_pad = _round_up(batch, _TILE)
    if batch_pad != batch:
        pad = batch_pad - batch
        u_idx = jnp.pad(u_idx, (0, pad))
        v_idx = jnp.pad(v_idx, (0, pad))
    n_tiles = batch_pad // _TILE

    u_ids = u_idx.reshape(n_tiles, 1, _TILE)
    v_ids = v_idx.reshape(n_tiles, 1, _TILE)
    u_t3 = u_table.reshape(nu, 1, emb)
    v_t3 = v_table.reshape(ni, 1, emb)

    out = pl.pallas_call(
        _gmf_gather_kernel,
        out_shape=jax.ShapeDtypeStruct((batch_pad, 1, emb), out_dtype),
        grid=(n_tiles,),
        in_specs=[
            pl.BlockSpec((1, 1, _TILE), lambda i: (i, 0, 0),
                         memory_space=pltpu.SMEM),
            pl.BlockSpec((1, 1, _TILE), lambda i: (i, 0, 0),
                         memory_space=pltpu.SMEM),
            pl.BlockSpec((nu, 1, emb), lambda i: (0, 0, 0)),
            pl.BlockSpec((ni, 1, emb), lambda i: (0, 0, 0)),
        ],
        out_specs=pl.BlockSpec((_TILE, 1, emb), lambda i: (i, 0, 0)),
        compiler_params=pltpu.CompilerParams(
            dimension_semantics=("parallel",),
            vmem_limit_bytes=56 * 1024 * 1024,
        ),
    )(u_ids, v_ids, u_t3, v_t3)

    return out.reshape(batch_pad, emb)[:batch]


# speedup vs baseline: 13.9997x; 1.6663x over previous
import jax
import jax.numpy as jnp
from jax.experimental import pallas as pl
from jax.experimental.pallas import tpu as pltpu

_TILE = 512


def _round_up(x: int, m: int) -> int:
    return (x + m - 1) // m * m


def _gmf_gather_kernel(u_ids_ref, v_ids_ref, u_tbl_ref, v_tbl_ref, out_ref):
    for mi in range(_TILE):
        u_row = u_tbl_ref[u_ids_ref[0, 0, mi], 0]
        v_row = v_tbl_ref[v_ids_ref[0, 0, mi], 0]
        out_ref[mi, 0] = u_row * v_row


@jax.jit
def kernel(u_idx, v_idx, u_table, v_table):
    batch = int(u_idx.shape[0])
    nu, emb = u_table.shape
    ni, emb_v = v_table.shape
    assert emb == emb_v, "embedding dims must match"
    out_dtype = jnp.result_type(u_table.dtype, v_table.dtype)

    u_idx = jnp.clip(u_idx.astype(jnp.int32), 0, nu - 1)
    v_idx = jnp.clip(v_idx.astype(jnp.int32), 0, ni - 1)

    batch_pad = _round_up(batch, _TILE)
    if batch_pad != batch:
        pad = batch_pad - batch
        u_idx = jnp.pad(u_idx, (0, pad))
        v_idx = jnp.pad(v_idx, (0, pad))
    n_tiles = batch_pad // _TILE

    u_ids = u_idx.reshape(n_tiles, 1, _TILE)
    v_ids = v_idx.reshape(n_tiles, 1, _TILE)
    u_t3 = u_table.reshape(nu, 1, emb)
    v_t3 = v_table.reshape(ni, 1, emb)

    out = pl.pallas_call(
        _gmf_gather_kernel,
        out_shape=jax.ShapeDtypeStruct((batch_pad, 1, emb), out_dtype),
        grid=(n_tiles,),
        in_specs=[
            pl.BlockSpec((1, 1, _TILE), lambda i: (i, 0, 0),
                         memory_space=pltpu.SMEM),
            pl.BlockSpec((1, 1, _TILE), lambda i: (i, 0, 0),
                         memory_space=pltpu.SMEM),
            pl.BlockSpec((nu, 1, emb), lambda i: (0, 0, 0)),
            pl.BlockSpec((ni, 1, emb), lambda i: (0, 0, 0)),
        ],
        out_specs=pl.BlockSpec((_TILE, 1, emb), lambda i: (i, 0, 0)),
        compiler_params=pltpu.CompilerParams(
            dimension_semantics=("parallel",),
            vmem_limit_bytes=56 * 1024 * 1024,
        ),
    )(u_ids, v_ids, u_t3, v_t3)

    return out.reshape(batch_pad, emb)[:batch]


# speedup vs baseline: 14.1757x; 1.0126x over previous
import jax
import jax.numpy as jnp
from jax.experimental import pallas as pl
from jax.experimental.pallas import tpu as pltpu

_TILE = 1024


def _round_up(x: int, m: int) -> int:
    return (x + m - 1) // m * m


def _gmf_gather_kernel(u_ids_ref, v_ids_ref, u_tbl_ref, v_tbl_ref, out_ref):
    for mi in range(_TILE):
        u_row = u_tbl_ref[u_ids_ref[0, 0, mi], 0]
        v_row = v_tbl_ref[v_ids_ref[0, 0, mi], 0]
        out_ref[mi, 0] = u_row * v_row


@jax.jit
def kernel(u_idx, v_idx, u_table, v_table):
    batch = int(u_idx.shape[0])
    nu, emb = u_table.shape
    ni, emb_v = v_table.shape
    assert emb == emb_v, "embedding dims must match"
    out_dtype = jnp.result_type(u_table.dtype, v_table.dtype)

    u_idx = jnp.clip(u_idx.astype(jnp.int32), 0, nu - 1)
    v_idx = jnp.clip(v_idx.astype(jnp.int32), 0, ni - 1)

    batch_pad = _round_up(batch, _TILE)
    if batch_pad != batch:
        pad = batch_pad - batch
        u_idx = jnp.pad(u_idx, (0, pad))
        v_idx = jnp.pad(v_idx, (0, pad))
    n_tiles = batch_pad // _TILE

    u_ids = u_idx.reshape(n_tiles, 1, _TILE)
    v_ids = v_idx.reshape(n_tiles, 1, _TILE)
    u_t3 = u_table.reshape(nu, 1, emb)
    v_t3 = v_table.reshape(ni, 1, emb)

    out = pl.pallas_call(
        _gmf_gather_kernel,
        out_shape=jax.ShapeDtypeStruct((batch_pad, 1, emb), out_dtype),
        grid=(n_tiles,),
        in_specs=[
            pl.BlockSpec((1, 1, _TILE), lambda i: (i, 0, 0),
                         memory_space=pltpu.SMEM),
            pl.BlockSpec((1, 1, _TILE), lambda i: (i, 0, 0),
                         memory_space=pltpu.SMEM),
            pl.BlockSpec((nu, 1, emb), lambda i: (0, 0, 0)),
            pl.BlockSpec((ni, 1, emb), lambda i: (0, 0, 0)),
        ],
        out_specs=pl.BlockSpec((_TILE, 1, emb), lambda i: (i, 0, 0)),
        compiler_params=pltpu.CompilerParams(
            dimension_semantics=("parallel",),
            vmem_limit_bytes=56 * 1024 * 1024,
        ),
    )(u_ids, v_ids, u_t3, v_t3)

    return out.reshape(batch_pad, emb)[:batch]
